# baseline (speedup 1.0000x reference)
import functools

import jax
import jax.numpy as jnp
from jax import lax
from jax.experimental import pallas as pl
from jax.experimental.pallas import tpu as pltpu
from jax.experimental.pallas import tpu_sc as plsc

NUM_SETS = 100000
EMBED_DIM = 64
HIDDEN_DIM = 128
NUM_CLASSES = 15
BATCH = 16384
TEAM = 6

NC = 2
NS = 16
NW = NC * NS
RW = BATCH // NW
CH = 16
NCH = RW // CH
IDX_PER_CH = CH * TEAM
IDX_PER_W = RW * TEAM


NBUF = 4


def _sc_pool_kernel(self_hbm, opp_hbm, emb_hbm, out_hbm,
                    idx_v, g0, g1, g2, g3, pool_v, s0, s1, s2, s3):
    gbufs = (g0, g1, g2, g3)
    sems = (s0, s1, s2, s3)
    wid = lax.axis_index("s") * NC + lax.axis_index("c")
    base = wid * RW

    def reduce_chunk(gbuf, c, col0):
        for i in range(CH):
            for d in range(EMBED_DIM // 16):
                sl = pl.ds(d * 16, 16)
                s = gbuf[i * TEAM, sl]
                for j in range(1, TEAM):
                    s = s + gbuf[i * TEAM + j, sl]
                pool_v[c * CH + i, pl.ds(col0 + d * 16, 16)] = s

    def gather_desc(c, b):
        return pltpu.make_async_copy(
            emb_hbm.at[idx_v.at[pl.ds(c * IDX_PER_CH, IDX_PER_CH)]],
            gbufs[b], sems[b])

    for t, team_hbm in enumerate((self_hbm, opp_hbm)):
        pltpu.sync_copy(team_hbm.at[pl.ds(base * TEAM, IDX_PER_W)], idx_v)
        col0 = t * EMBED_DIM

        for b in range(NBUF):
            gather_desc(b, b).start()

        def ring_body(q, carry):
            for b in range(NBUF):
                c = q * NBUF + b
                gather_desc(c, b).wait()
                reduce_chunk(gbufs[b], c, col0)

                @pl.when(q < NCH // NBUF - 1)
                def _():
                    gather_desc(c + NBUF, b).start()
            return carry

        lax.fori_loop(0, NCH // NBUF, ring_body, 0)

    pltpu.sync_copy(pool_v, out_hbm.at[pl.ds(base, RW)])


def _sc_pool(self_idx, opp_idx, embedding):
    mesh = plsc.VectorSubcoreMesh(core_axis_name="c", subcore_axis_name="s",
                                  num_cores=NC, num_subcores=NS)
    f = functools.partial(
        pl.kernel,
        out_type=jax.ShapeDtypeStruct((BATCH, 2 * EMBED_DIM), jnp.float32),
        mesh=mesh,
        compiler_params=pltpu.CompilerParams(use_tc_tiling_on_sc=False),
        scratch_types=(
            [pltpu.VMEM((IDX_PER_W,), jnp.int32)]
            + [pltpu.VMEM((IDX_PER_CH, EMBED_DIM), jnp.float32)] * NBUF
            + [pltpu.VMEM((RW, 2 * EMBED_DIM), jnp.float32)]
            + [pltpu.SemaphoreType.DMA] * NBUF
        ),
    )(_sc_pool_kernel)
    return f(self_idx, opp_idx, embedding)


def _mlp_kernel(x_ref, w1_ref, b1_ref, w2t_ref, b2_ref, out_ref):
    w1t = jnp.transpose(w1_ref[...]) * (1.0 / TEAM)
    h = jnp.dot(x_ref[...], w1t, preferred_element_type=jnp.float32,
                precision=lax.Precision.HIGHEST) + b1_ref[...]
    h = jnp.maximum(h, 0.0)
    out_ref[...] = (jnp.dot(h, w2t_ref[...], preferred_element_type=jnp.float32,
                            precision=lax.Precision.HIGHEST)
                    + b2_ref[...])


def _mlp(pooled, W1, b1, W2, b2):
    blk = 2048
    grid = (BATCH // blk,)
    return pl.pallas_call(
        _mlp_kernel,
        grid=grid,
        in_specs=[
            pl.BlockSpec((blk, 2 * EMBED_DIM), lambda i: (i, 0)),
            pl.BlockSpec((HIDDEN_DIM, 2 * EMBED_DIM), lambda i: (0, 0)),
            pl.BlockSpec((1, HIDDEN_DIM), lambda i: (0, 0)),
            pl.BlockSpec((HIDDEN_DIM, NUM_CLASSES), lambda i: (0, 0)),
            pl.BlockSpec((1, NUM_CLASSES), lambda i: (0, 0)),
        ],
        out_specs=pl.BlockSpec((blk, NUM_CLASSES), lambda i: (i, 0)),
        out_shape=jax.ShapeDtypeStruct((BATCH, NUM_CLASSES), jnp.float32),
    )(pooled, W1, b1.reshape(1, HIDDEN_DIM), W2.T, b2.reshape(1, NUM_CLASSES))


def kernel(self_team, opp_team, embedding, W1, b1, W2, b2):
    self_idx = self_team.astype(jnp.int32).reshape(BATCH * TEAM)
    opp_idx = opp_team.astype(jnp.int32).reshape(BATCH * TEAM)
    pooled = _sc_pool(self_idx, opp_idx, embedding)
    return _mlp(pooled, W1, b1, W2, b2)


# speedup vs baseline: 1.0824x; 1.0824x over previous
import functools

import jax
import jax.numpy as jnp
from jax import lax
from jax.experimental import pallas as pl
from jax.experimental.pallas import tpu as pltpu
from jax.experimental.pallas import tpu_sc as plsc

NUM_SETS = 100000
EMBED_DIM = 64
HIDDEN_DIM = 128
NUM_CLASSES = 15
BATCH = 16384
TEAM = 6

NC = 2
NS = 16
NW = NC * NS
NHALF = 2
BH = BATCH // NHALF
RW = BH // NW
CH = 16
NCH = RW // CH
IDX_PER_CH = CH * TEAM
IDX_PER_W = RW * TEAM


def _make_sc_kernel(half):
    def _sc_pool_kernel(self_hbm, opp_hbm, emb_hbm, out_hbm,
                        idx_v, gbuf0, gbuf1, pool_v, sem0, sem1):
        wid = lax.axis_index("s") * NC + lax.axis_index("c")
        base = half * BH + wid * RW

        def reduce_chunk(gbuf, c, col0):
            for i in range(CH):
                for d in range(EMBED_DIM // 16):
                    sl = pl.ds(d * 16, 16)
                    s = gbuf[i * TEAM, sl]
                    for j in range(1, TEAM):
                        s = s + gbuf[i * TEAM + j, sl]
                    pool_v[c * CH + i, pl.ds(col0 + d * 16, 16)] = s

        def gather_desc(c, gbuf, sem):
            return pltpu.make_async_copy(
                emb_hbm.at[idx_v.at[pl.ds(c * IDX_PER_CH, IDX_PER_CH)]],
                gbuf, sem)

        for t, team_hbm in enumerate((self_hbm, opp_hbm)):
            pltpu.sync_copy(team_hbm.at[pl.ds(base * TEAM, IDX_PER_W)], idx_v)
            col0 = t * EMBED_DIM

            gather_desc(0, gbuf0, sem0).start()

            def pair_body(i, carry):
                c0 = 2 * i
                gather_desc(c0 + 1, gbuf1, sem1).start()
                gather_desc(c0, gbuf0, sem0).wait()
                reduce_chunk(gbuf0, c0, col0)

                @pl.when(i < NCH // 2 - 1)
                def _():
                    gather_desc(c0 + 2, gbuf0, sem0).start()

                gather_desc(c0 + 1, gbuf1, sem1).wait()
                reduce_chunk(gbuf1, c0 + 1, col0)
                return carry

            lax.fori_loop(0, NCH // 2, pair_body, 0)

        pltpu.sync_copy(pool_v, out_hbm.at[pl.ds(wid * RW, RW)])

    return _sc_pool_kernel


def _sc_pool(self_idx, opp_idx, embedding, half):
    mesh = plsc.VectorSubcoreMesh(core_axis_name="c", subcore_axis_name="s",
                                  num_cores=NC, num_subcores=NS)
    f = functools.partial(
        pl.kernel,
        out_type=jax.ShapeDtypeStruct((BH, 2 * EMBED_DIM), jnp.float32),
        mesh=mesh,
        compiler_params=pltpu.CompilerParams(use_tc_tiling_on_sc=False),
        scratch_types=[
            pltpu.VMEM((IDX_PER_W,), jnp.int32),
            pltpu.VMEM((IDX_PER_CH, EMBED_DIM), jnp.float32),
            pltpu.VMEM((IDX_PER_CH, EMBED_DIM), jnp.float32),
            pltpu.VMEM((RW, 2 * EMBED_DIM), jnp.float32),
            pltpu.SemaphoreType.DMA,
            pltpu.SemaphoreType.DMA,
        ],
        name=f"sc_pool_h{half}",
    )(_make_sc_kernel(half))
    return f(self_idx, opp_idx, embedding)


def _mlp_kernel(x_ref, w1_ref, b1_ref, w2t_ref, b2_ref, out_ref):
    w1t = jnp.transpose(w1_ref[...]) * (1.0 / TEAM)
    h = jnp.dot(x_ref[...], w1t, preferred_element_type=jnp.float32,
                precision=lax.Precision.HIGHEST) + b1_ref[...]
    h = jnp.maximum(h, 0.0)
    out_ref[...] = (jnp.dot(h, w2t_ref[...], preferred_element_type=jnp.float32,
                            precision=lax.Precision.HIGHEST)
                    + b2_ref[...])


def _mlp(pooled, W1, b1, W2t, b2):
    blk = 2048
    grid = (BH // blk,)
    return pl.pallas_call(
        _mlp_kernel,
        grid=grid,
        in_specs=[
            pl.BlockSpec((blk, 2 * EMBED_DIM), lambda i: (i, 0)),
            pl.BlockSpec((HIDDEN_DIM, 2 * EMBED_DIM), lambda i: (0, 0)),
            pl.BlockSpec((1, HIDDEN_DIM), lambda i: (0, 0)),
            pl.BlockSpec((HIDDEN_DIM, NUM_CLASSES), lambda i: (0, 0)),
            pl.BlockSpec((1, NUM_CLASSES), lambda i: (0, 0)),
        ],
        out_specs=pl.BlockSpec((blk, NUM_CLASSES), lambda i: (i, 0)),
        out_shape=jax.ShapeDtypeStruct((BH, NUM_CLASSES), jnp.float32),
    )(pooled, W1, b1, W2t, b2)


def kernel(self_team, opp_team, embedding, W1, b1, W2, b2):
    self_idx = self_team.astype(jnp.int32).reshape(BATCH * TEAM)
    opp_idx = opp_team.astype(jnp.int32).reshape(BATCH * TEAM)
    b1r = b1.reshape(1, HIDDEN_DIM)
    W2t = W2.T
    b2r = b2.reshape(1, NUM_CLASSES)
    logits = []
    for half in range(NHALF):
        pooled = _sc_pool(self_idx, opp_idx, embedding, half)
        logits.append(_mlp(pooled, W1, b1r, W2t, b2r))
    return jnp.concatenate(logits, axis=0)
